# baseline (speedup 1.0000x reference)
import jax
import jax.numpy as jnp
from jax.experimental import pallas as pl

NX = 2048
T_OUT = 50
T_IN = 10
STATE = 1
PSIZE = 5
WIDTH = 64
DEPTH = 4
B = 1

CHUNK = 512
HALO = DEPTH
GRID = NX // CHUNK
COLS = CHUNK + 2 * HALO
NF = T_OUT * COLS


def _interp_kernel(u0_ref, out_ref):
    k = jax.lax.broadcasted_iota(jnp.int32, (T_IN, T_OUT), 0)
    t = jax.lax.broadcasted_iota(jnp.int32, (T_IN, T_OUT), 1)
    xs = jnp.maximum((t.astype(jnp.float32) + 0.5) * (T_IN / T_OUT) - 0.5, 0.0)
    x0 = jnp.floor(xs)
    wx = xs - x0
    x0i = x0.astype(jnp.int32)
    x1i = jnp.minimum(x0i + 1, T_IN - 1)
    M = jnp.where(k == x0i, 1.0 - wx, 0.0) + jnp.where(k == x1i, wx, 0.0)
    out_ref[...] = jnp.dot(u0_ref[...], M, preferred_element_type=jnp.float32)


def _gcn_kernel(u0i_ref, P_ref, fciW_ref, fcib_ref, W_ref, b_ref, lg_ref,
                lb_ref, fcoW_ref, fcob_ref, out_ref):
    p = pl.program_id(0)
    f32 = jnp.float32

    uf = u0i_ref[0]

    j = jax.lax.broadcasted_iota(jnp.int32, (1, NF), 1)
    ti = j // COLS
    gi = p * CHUNK + (j - ti * COLS) - HALO
    n = ti * NX + gi
    i_f = n // T_OUT
    t_f = n - i_f * T_OUT
    xgv = i_f.astype(f32) * (1.0 / (NX - 1))
    tgv = t_f.astype(f32) * (1.0 / (T_OUT - 1))

    Wfi = fciW_ref[...]
    base = fcib_ref[...] + jax.lax.dot_general(
        P_ref[...], Wfi[:, 1:1 + PSIZE],
        (((1,), (1,)), ((), ())), preferred_element_type=f32)

    S3 = jnp.concatenate([uf, xgv, tgv], axis=0)
    Wsel = jnp.concatenate(
        [Wfi[:, 0:1], Wfi[:, STATE + PSIZE:STATE + PSIZE + 1],
         Wfi[:, STATE + PSIZE + 1:STATE + PSIZE + 2]], axis=1)
    x = jax.lax.dot_general(
        Wsel, S3, (((1,), (0,)), ((), ())),
        preferred_element_type=f32) + base.reshape(WIDTH, 1)

    inb = jnp.logical_and(gi >= 0, gi <= NX - 1)
    deg = (1.0
           + jnp.where(gi > 0, 1.0, 0.0) + jnp.where(gi < NX - 1, 1.0, 0.0)
           + jnp.where(ti > 0, 1.0, 0.0) + jnp.where(ti < T_OUT - 1, 1.0, 0.0))
    dinv = jnp.where(inb, jax.lax.rsqrt(deg), 0.0)

    ones_w = jnp.full((1, WIDTH), 1.0 / WIDTH, f32)

    def stencil(a):
        R = a.shape[0]
        s = a
        s = s + jnp.concatenate([jnp.zeros((R, 1), f32), a[:, :-1]], axis=1)
        s = s + jnp.concatenate([a[:, 1:], jnp.zeros((R, 1), f32)], axis=1)
        s = s + jnp.concatenate([jnp.zeros((R, COLS), f32), a[:, :-COLS]],
                                axis=1)
        s = s + jnp.concatenate([a[:, COLS:], jnp.zeros((R, COLS), f32)],
                                axis=1)
        return s

    for l in range(DEPTH):
        h = jax.lax.dot_general(
            W_ref[l], x, (((1,), (0,)), ((), ())),
            preferred_element_type=f32)
        g = h * dinv
        ha = stencil(g) * dinv + b_ref[l].reshape(WIDTH, 1)
        hbar = jax.lax.dot_general(
            ones_w, h, (((1,), (0,)), ((), ())),
            preferred_element_type=f32)
        m = stencil(hbar * dinv) * dinv + jnp.mean(b_ref[l])
        q = jax.lax.dot_general(
            ones_w, ha * ha, (((1,), (0,)), ((), ())),
            preferred_element_type=f32)
        r = jax.lax.rsqrt(q - m * m + 1e-5)
        h = (ha - m) * r * lg_ref[l].reshape(WIDTH, 1) \
            + lb_ref[l].reshape(WIDTH, 1)
        x = jnp.maximum(h + x, 0.0)

    o = jax.lax.dot_general(
        fcoW_ref[...], x, (((1,), (0,)), ((), ())),
        preferred_element_type=f32) + fcob_ref[0, 0]
    for t in range(T_OUT):
        out_ref[t:t + 1, :] = o[0:1, t * COLS + HALO:t * COLS + HALO + CHUNK]


@jax.jit
def kernel(u0, P, fc_in_W, fc_in_b, gcn_W, gcn_b, ln_g, ln_b, fc_out_W,
           fc_out_b):
    u0_2d = u0.reshape(NX, T_IN)
    u0i = pl.pallas_call(
        _interp_kernel,
        out_shape=jax.ShapeDtypeStruct((NX, T_OUT), jnp.float32),
    )(u0_2d)
    u0i_node = u0i.reshape(T_OUT, NX)
    u0i_pad = jnp.pad(u0i_node, ((0, 0), (HALO, HALO)))
    u0i_im = jnp.stack([
        u0i_pad[:, p * CHUNK:p * CHUNK + COLS].reshape(1, NF)
        for p in range(GRID)])

    Wl = jnp.stack(gcn_W)
    bl = jnp.stack(gcn_b)
    lgl = jnp.stack(ln_g)
    lbl = jnp.stack(ln_b)
    fcib = fc_in_b.reshape(1, WIDTH)
    fcob = fc_out_b.reshape(1, 1)

    full = lambda a: pl.BlockSpec(a.shape, lambda p: (0,) * a.ndim)
    out2d = pl.pallas_call(
        _gcn_kernel,
        grid=(GRID,),
        in_specs=[pl.BlockSpec((1, 1, NF), lambda p: (p, 0, 0)),
                  full(P), full(fc_in_W), full(fcib),
                  full(Wl), full(bl), full(lgl), full(lbl),
                  full(fc_out_W), full(fcob)],
        out_specs=pl.BlockSpec((T_OUT, CHUNK), lambda p: (0, p)),
        out_shape=jax.ShapeDtypeStruct((T_OUT, NX), jnp.float32),
    )(u0i_im, P, fc_in_W, fcib, Wl, bl, lgl, lbl, fc_out_W, fcob)

    return out2d.reshape(B, NX, T_OUT, STATE)


# speedup vs baseline: 1.0239x; 1.0239x over previous
import jax
import jax.numpy as jnp
from jax.experimental import pallas as pl

NX = 2048
T_OUT = 50
T_IN = 10
STATE = 1
PSIZE = 5
WIDTH = 64
DEPTH = 4
B = 1

CHUNK = 512
HALO = DEPTH
GRID = NX // CHUNK
COLS = CHUNK + 2 * HALO
NF = T_OUT * COLS


def _interp_kernel(u0_ref, out_ref):
    k = jax.lax.broadcasted_iota(jnp.int32, (T_IN, T_OUT), 0)
    t = jax.lax.broadcasted_iota(jnp.int32, (T_IN, T_OUT), 1)
    xs = jnp.maximum((t.astype(jnp.float32) + 0.5) * (T_IN / T_OUT) - 0.5, 0.0)
    x0 = jnp.floor(xs)
    wx = xs - x0
    x0i = x0.astype(jnp.int32)
    x1i = jnp.minimum(x0i + 1, T_IN - 1)
    M = jnp.where(k == x0i, 1.0 - wx, 0.0) + jnp.where(k == x1i, wx, 0.0)
    out_ref[...] = jnp.dot(u0_ref[...], M, preferred_element_type=jnp.float32)


def _gcn_kernel(u0i_ref, P_ref, fciW_ref, fcib_ref, W_ref, b_ref, lg_ref,
                lb_ref, fcoW_ref, fcob_ref, out_ref):
    p = pl.program_id(0)
    f32 = jnp.float32

    uf = u0i_ref[0]

    j = jax.lax.broadcasted_iota(jnp.int32, (1, NF), 1)
    ti = j // COLS
    gi = p * CHUNK + (j - ti * COLS) - HALO
    n = ti * NX + gi
    i_f = n // T_OUT
    t_f = n - i_f * T_OUT
    xgv = i_f.astype(f32) * (1.0 / (NX - 1))
    tgv = t_f.astype(f32) * (1.0 / (T_OUT - 1))

    Wfi = fciW_ref[...]
    base = fcib_ref[...] + jax.lax.dot_general(
        P_ref[...], Wfi[:, 1:1 + PSIZE],
        (((1,), (1,)), ((), ())), preferred_element_type=f32)

    x = (uf * Wfi[:, 0:1]
         + xgv * Wfi[:, STATE + PSIZE:STATE + PSIZE + 1]
         + tgv * Wfi[:, STATE + PSIZE + 1:STATE + PSIZE + 2]
         + base.reshape(WIDTH, 1))

    inb = jnp.logical_and(gi >= 0, gi <= NX - 1)
    deg = (1.0
           + jnp.where(gi > 0, 1.0, 0.0) + jnp.where(gi < NX - 1, 1.0, 0.0)
           + jnp.where(ti > 0, 1.0, 0.0) + jnp.where(ti < T_OUT - 1, 1.0, 0.0))
    dinv = jnp.where(inb, jax.lax.rsqrt(deg), 0.0)

    zc = jnp.zeros((WIDTH, COLS), f32)
    for l in range(DEPTH):
        h = jax.lax.dot_general(
            W_ref[l], x, (((1,), (0,)), ((), ())),
            preferred_element_type=f32)
        g = h * dinv
        s = g
        s = s + jnp.roll(g, 1, axis=1)
        s = s + jnp.roll(g, -1, axis=1)
        s = s + jnp.concatenate([zc, g[:, :-COLS]], axis=1)
        s = s + jnp.concatenate([g[:, COLS:], zc], axis=1)
        h = s * dinv + b_ref[l].reshape(WIDTH, 1)
        m = jnp.mean(h, axis=0, keepdims=True)
        d = h - m
        v = jnp.mean(d * d, axis=0, keepdims=True)
        h = d * jax.lax.rsqrt(v + 1e-5) * lg_ref[l].reshape(WIDTH, 1) \
            + lb_ref[l].reshape(WIDTH, 1)
        x = jnp.maximum(h + x, 0.0)

    o = jnp.sum(x * fcoW_ref[0].reshape(WIDTH, 1), axis=0,
                keepdims=True) + fcob_ref[0, 0]
    for t in range(T_OUT):
        out_ref[t:t + 1, :] = o[0:1, t * COLS + HALO:t * COLS + HALO + CHUNK]


@jax.jit
def kernel(u0, P, fc_in_W, fc_in_b, gcn_W, gcn_b, ln_g, ln_b, fc_out_W,
           fc_out_b):
    u0_2d = u0.reshape(NX, T_IN)
    u0i = pl.pallas_call(
        _interp_kernel,
        out_shape=jax.ShapeDtypeStruct((NX, T_OUT), jnp.float32),
    )(u0_2d)
    u0i_node = u0i.reshape(T_OUT, NX)
    u0i_pad = jnp.pad(u0i_node, ((0, 0), (HALO, HALO)))
    u0i_im = jnp.stack([
        u0i_pad[:, p * CHUNK:p * CHUNK + COLS].reshape(1, NF)
        for p in range(GRID)])

    Wl = jnp.stack(gcn_W)
    bl = jnp.stack(gcn_b)
    lgl = jnp.stack(ln_g)
    lbl = jnp.stack(ln_b)
    fcib = fc_in_b.reshape(1, WIDTH)
    fcob = fc_out_b.reshape(1, 1)

    full = lambda a: pl.BlockSpec(a.shape, lambda p: (0,) * a.ndim)
    out2d = pl.pallas_call(
        _gcn_kernel,
        grid=(GRID,),
        in_specs=[pl.BlockSpec((1, 1, NF), lambda p: (p, 0, 0)),
                  full(P), full(fc_in_W), full(fcib),
                  full(Wl), full(bl), full(lgl), full(lbl),
                  full(fc_out_W), full(fcob)],
        out_specs=pl.BlockSpec((T_OUT, CHUNK), lambda p: (0, p)),
        out_shape=jax.ShapeDtypeStruct((T_OUT, NX), jnp.float32),
    )(u0i_im, P, fc_in_W, fcib, Wl, bl, lgl, lbl, fc_out_W, fcob)

    return out2d.reshape(B, NX, T_OUT, STATE)


# speedup vs baseline: 1.2282x; 1.1995x over previous
import jax
import jax.numpy as jnp
from jax.experimental import pallas as pl

NX = 2048
T_OUT = 50
T_IN = 10
STATE = 1
PSIZE = 5
WIDTH = 64
DEPTH = 4
B = 1

CHUNK = 256
HALO = DEPTH
GRID = NX // CHUNK
COLS = CHUNK + 2 * HALO
NF = T_OUT * COLS


def _interp_kernel(u0_ref, out_ref):
    k = jax.lax.broadcasted_iota(jnp.int32, (T_IN, T_OUT), 0)
    t = jax.lax.broadcasted_iota(jnp.int32, (T_IN, T_OUT), 1)
    xs = jnp.maximum((t.astype(jnp.float32) + 0.5) * (T_IN / T_OUT) - 0.5, 0.0)
    x0 = jnp.floor(xs)
    wx = xs - x0
    x0i = x0.astype(jnp.int32)
    x1i = jnp.minimum(x0i + 1, T_IN - 1)
    M = jnp.where(k == x0i, 1.0 - wx, 0.0) + jnp.where(k == x1i, wx, 0.0)
    out_ref[...] = jnp.dot(u0_ref[...], M, preferred_element_type=jnp.float32)


def _gcn_kernel(u0i_ref, P_ref, fciW_ref, fcib_ref, W_ref, b_ref, lg_ref,
                lb_ref, fcoW_ref, fcob_ref, out_ref):
    p = pl.program_id(0)
    f32 = jnp.float32

    uf = u0i_ref[0]

    j = jax.lax.broadcasted_iota(jnp.int32, (1, NF), 1)
    ti = j // COLS
    gi = p * CHUNK + (j - ti * COLS) - HALO
    n = ti * NX + gi
    i_f = n // T_OUT
    t_f = n - i_f * T_OUT
    xgv = i_f.astype(f32) * (1.0 / (NX - 1))
    tgv = t_f.astype(f32) * (1.0 / (T_OUT - 1))

    Wfi = fciW_ref[...]
    base = fcib_ref[...] + jax.lax.dot_general(
        P_ref[...], Wfi[:, 1:1 + PSIZE],
        (((1,), (1,)), ((), ())), preferred_element_type=f32)

    x = (uf * Wfi[:, 0:1]
         + xgv * Wfi[:, STATE + PSIZE:STATE + PSIZE + 1]
         + tgv * Wfi[:, STATE + PSIZE + 1:STATE + PSIZE + 2]
         + base.reshape(WIDTH, 1))

    inb = jnp.logical_and(gi >= 0, gi <= NX - 1)
    deg = (1.0
           + jnp.where(gi > 0, 1.0, 0.0) + jnp.where(gi < NX - 1, 1.0, 0.0)
           + jnp.where(ti > 0, 1.0, 0.0) + jnp.where(ti < T_OUT - 1, 1.0, 0.0))
    dinv = jnp.where(inb, jax.lax.rsqrt(deg), 0.0)

    zc = jnp.zeros((WIDTH, COLS), f32)
    for l in range(DEPTH):
        h = jax.lax.dot_general(
            W_ref[l], x, (((1,), (0,)), ((), ())),
            preferred_element_type=f32)
        g = h * dinv
        s = g
        s = s + jnp.roll(g, 1, axis=1)
        s = s + jnp.roll(g, -1, axis=1)
        s = s + jnp.concatenate([zc, g[:, :-COLS]], axis=1)
        s = s + jnp.concatenate([g[:, COLS:], zc], axis=1)
        h = s * dinv + b_ref[l].reshape(WIDTH, 1)
        m = jnp.mean(h, axis=0, keepdims=True)
        d = h - m
        v = jnp.mean(d * d, axis=0, keepdims=True)
        h = d * jax.lax.rsqrt(v + 1e-5) * lg_ref[l].reshape(WIDTH, 1) \
            + lb_ref[l].reshape(WIDTH, 1)
        x = jnp.maximum(h + x, 0.0)

    o = jnp.sum(x * fcoW_ref[0].reshape(WIDTH, 1), axis=0,
                keepdims=True) + fcob_ref[0, 0]
    for t in range(T_OUT):
        out_ref[t:t + 1, :] = o[0:1, t * COLS + HALO:t * COLS + HALO + CHUNK]


@jax.jit
def kernel(u0, P, fc_in_W, fc_in_b, gcn_W, gcn_b, ln_g, ln_b, fc_out_W,
           fc_out_b):
    u0_2d = u0.reshape(NX, T_IN)
    u0i = pl.pallas_call(
        _interp_kernel,
        out_shape=jax.ShapeDtypeStruct((NX, T_OUT), jnp.float32),
    )(u0_2d)
    u0i_node = u0i.reshape(T_OUT, NX)
    u0i_pad = jnp.pad(u0i_node, ((0, 0), (HALO, HALO)))
    u0i_im = jnp.stack([
        u0i_pad[:, p * CHUNK:p * CHUNK + COLS].reshape(1, NF)
        for p in range(GRID)])

    Wl = jnp.stack(gcn_W)
    bl = jnp.stack(gcn_b)
    lgl = jnp.stack(ln_g)
    lbl = jnp.stack(ln_b)
    fcib = fc_in_b.reshape(1, WIDTH)
    fcob = fc_out_b.reshape(1, 1)

    full = lambda a: pl.BlockSpec(a.shape, lambda p: (0,) * a.ndim)
    out2d = pl.pallas_call(
        _gcn_kernel,
        grid=(GRID,),
        in_specs=[pl.BlockSpec((1, 1, NF), lambda p: (p, 0, 0)),
                  full(P), full(fc_in_W), full(fcib),
                  full(Wl), full(bl), full(lgl), full(lbl),
                  full(fc_out_W), full(fcob)],
        out_specs=pl.BlockSpec((T_OUT, CHUNK), lambda p: (0, p)),
        out_shape=jax.ShapeDtypeStruct((T_OUT, NX), jnp.float32),
    )(u0i_im, P, fc_in_W, fcib, Wl, bl, lgl, lbl, fc_out_W, fcob)

    return out2d.reshape(B, NX, T_OUT, STATE)
